# baseline (speedup 1.0000x reference)
import functools

import jax
import jax.numpy as jnp
from jax import lax
from jax.experimental import pallas as pl
from jax.experimental.pallas import tpu as pltpu
from jax.experimental.pallas import tpu_sc as plsc

_RADIUS = 0.033
_K = 32
_G = 30
_NCELL = _G * _G
_CAP = 24
_NCELL_PAD = 912
_NQ = 16384
_NL = 4096
_NW = 32
_QPW = _NQ // _NW


_CH = 128


def _search_body(latx_hbm, laty_hbm, qx_hbm, qy_hbm, rn_hbm,
                 nbr_hbm, ynx_hbm, yny_hbm, cnt_hbm, fnbr_hbm,
                 latx, laty, ccnt, table, qx, qy,
                 nbrb, ynxb, ynyb, cntb, rows0, rows1, idx0, idx1,
                 sem0, sem1):
    wid = lax.axis_index("s") * 2 + lax.axis_index("c")
    qbase = wid * _QPW
    pltpu.sync_copy(latx_hbm, latx)
    pltpu.sync_copy(laty_hbm, laty)
    pltpu.sync_copy(qx_hbm.at[pl.ds(qbase, _QPW)], qx)
    pltpu.sync_copy(qy_hbm.at[pl.ds(qbase, _QPW)], qy)

    zi = jnp.zeros((16,), jnp.int32)
    zf = jnp.zeros((16,), jnp.float32)

    def zero_ccnt(i, _):
        ccnt[pl.ds(i * 16, 16)] = zi
        return 0
    lax.fori_loop(0, _NCELL_PAD // 16, zero_ccnt, 0)

    def zero_bufs(i, _):
        nbrb[pl.ds(i * 16, 16)] = zi
        ynxb[pl.ds(i * 16, 16)] = zf
        ynyb[pl.ds(i * 16, 16)] = zf
        return 0
    lax.fori_loop(0, (_QPW * _K) // 16, zero_bufs, 0)

    lane = lax.broadcasted_iota(jnp.int32, (16,), 0)

    def build16(i, _):
        xv = latx[pl.ds(i * 16, 16)]
        yv = laty[pl.ds(i * 16, 16)]
        cx = (xv * _G).astype(jnp.int32)
        cy = (yv * _G).astype(jnp.int32)
        c = cy * _G + cx
        idxv = i * 16 + lane
        for j in range(16):
            m = lane == j
            cur = plsc.load_gather(ccnt, [c], mask=m)
            ok = m & (cur < _CAP)
            slot = jnp.minimum(cur, _CAP - 1)
            plsc.store_scatter(table, [c * _CAP + slot], idxv, mask=ok)
            plsc.store_scatter(ccnt, [c], cur + 1, mask=ok)
        return 0
    lax.fori_loop(0, _NL // 16, build16, 0)

    r2 = jnp.float32(_RADIUS * _RADIUS)

    def group(g, _):
        qxv = qx[pl.ds(g * 16, 16)]
        qyv = qy[pl.ds(g * 16, 16)]
        cx = (qxv * _G).astype(jnp.int32)
        cy = (qyv * _G).astype(jnp.int32)
        base = (g * 16 + lane) * _K
        nc = jnp.zeros((16,), jnp.int32)
        for dy in (-1, 0, 1):
            for dx in (-1, 0, 1):
                ny = cy + dy
                nx = cx + dx
                cellok = (ny >= 0) & (ny < _G) & (nx >= 0) & (nx < _G)
                cidx = jnp.where(cellok, ny * _G + nx, 0)
                ccv = plsc.load_gather(ccnt, [cidx], mask=cellok)
                ccv = jnp.where(cellok, ccv, 0)
                mx = jnp.max(ccv)

                def entry(e, nc):
                    valid = e < ccv
                    v = plsc.load_gather(table, [cidx * _CAP + e], mask=valid)
                    lx = plsc.load_gather(latx, [v], mask=valid)
                    ly = plsc.load_gather(laty, [v], mask=valid)
                    d0 = qxv - lx
                    d1 = qyv - ly
                    dd = d0 * d0 + d1 * d1
                    hit = valid & (dd <= r2)
                    ok = hit & (nc < _K)
                    addr = base + jnp.minimum(nc, _K - 1)
                    plsc.store_scatter(nbrb, [addr], v, mask=ok)
                    plsc.store_scatter(ynxb, [addr], lx, mask=ok)
                    plsc.store_scatter(ynyb, [addr], ly, mask=ok)
                    return nc + ok.astype(jnp.int32)

                nc = lax.fori_loop(0, mx, entry, nc)
        cntb[pl.ds(g * 16, 16)] = nc.astype(jnp.float32)
        return 0
    lax.fori_loop(0, _QPW // 16, group, 0)

    pltpu.sync_copy(nbrb, nbr_hbm.at[pl.ds(qbase * _K, _QPW * _K)])
    pltpu.sync_copy(ynxb, ynx_hbm.at[pl.ds(qbase * _K, _QPW * _K)])
    pltpu.sync_copy(ynyb, yny_hbm.at[pl.ds(qbase * _K, _QPW * _K)])
    pltpu.sync_copy(cntb, cnt_hbm.at[pl.ds(qbase, _QPW)])

    ebase = qbase * _K
    nch = (_QPW * _K) // _CH
    rows = (rows0, rows1)
    idxs = (idx0, idx1)
    sems = (sem0, sem1)
    for v in range(_CH // 16):
        idx0[pl.ds(v * 16, 16)] = nbrb[pl.ds(v * 16, 16)]
    prev = None
    for c in range(nch):
        buf = rows[c % 2]
        cp = pltpu.async_copy(rn_hbm.at[idxs[c % 2]], buf, sems[c % 2])
        if prev is not None:
            pcp, pbuf, poff = prev
            pcp.wait()
            pltpu.sync_copy(pbuf, fnbr_hbm.at[pl.ds(ebase + poff, _CH)])
        if c + 1 < nch:
            nidx = idxs[(c + 1) % 2]
            off = (c + 1) * _CH
            for v in range(_CH // 16):
                nidx[pl.ds(v * 16, 16)] = nbrb[pl.ds(off + v * 16, 16)]
        prev = (cp, buf, c * _CH)
    pcp, pbuf, poff = prev
    pcp.wait()
    pltpu.sync_copy(pbuf, fnbr_hbm.at[pl.ds(ebase + poff, _CH)])


def _sc_search(latx, laty, qx, qy, rndata_flat):
    mesh = plsc.VectorSubcoreMesh(core_axis_name="c", subcore_axis_name="s")
    f = pl.kernel(
        _search_body,
        out_type=[
            jax.ShapeDtypeStruct((_NQ * _K,), jnp.int32),
            jax.ShapeDtypeStruct((_NQ * _K,), jnp.float32),
            jax.ShapeDtypeStruct((_NQ * _K,), jnp.float32),
            jax.ShapeDtypeStruct((_NQ,), jnp.float32),
            jax.ShapeDtypeStruct((_NQ * _K, 128), jnp.float32),
        ],
        mesh=mesh,
        compiler_params=pltpu.CompilerParams(needs_layout_passes=False),
        scratch_types=[
            pltpu.VMEM((_NL,), jnp.float32),
            pltpu.VMEM((_NL,), jnp.float32),
            pltpu.VMEM((_NCELL_PAD,), jnp.int32),
            pltpu.VMEM((_NCELL * _CAP,), jnp.int32),
            pltpu.VMEM((_QPW,), jnp.float32),
            pltpu.VMEM((_QPW,), jnp.float32),
            pltpu.VMEM((_QPW * _K,), jnp.int32),
            pltpu.VMEM((_QPW * _K,), jnp.float32),
            pltpu.VMEM((_QPW * _K,), jnp.float32),
            pltpu.VMEM((_QPW,), jnp.float32),
            pltpu.VMEM((_CH, 128), jnp.float32),
            pltpu.VMEM((_CH, 128), jnp.float32),
            pltpu.VMEM((_CH,), jnp.int32),
            pltpu.VMEM((_CH,), jnp.int32),
            pltpu.SemaphoreType.DMA,
            pltpu.SemaphoreType.DMA,
        ],
    )
    return f(latx, laty, qx, qy, rndata_flat)


def _mlp_block(kin_ref, fnbr_ref, cntE_ref, cntQ_ref,
               W0_ref, b0_ref, W1_ref, b1_ref, W2_ref, b2_ref,
               Wp1_ref, bp1_ref, Wp2_ref, bp2_ref, out_ref):
    E = kin_ref.shape[0]
    Bq = E // _K
    h = jax.nn.gelu(jnp.dot(kin_ref[...], W0_ref[...],
                            preferred_element_type=jnp.float32) + b0_ref[...][None, :])
    h = jax.nn.gelu(jnp.dot(h, W1_ref[...],
                            preferred_element_type=jnp.float32) + b1_ref[...][None, :])
    kern = jnp.dot(h, W2_ref[...],
                   preferred_element_type=jnp.float32) + b2_ref[...][None, :]
    k2d = jax.lax.broadcasted_iota(jnp.int32, (E, 128), 0) % _K
    cntEb = jnp.broadcast_to(cntE_ref[...], (E, 128)).astype(jnp.int32)
    maskE = (k2d < cntEb).astype(jnp.float32)
    prod = kern * fnbr_ref[...] * maskE
    summed = prod.reshape(Bq, _K, 128).sum(axis=1)
    cnt = jnp.maximum(cntQ_ref[...], 1.0)
    decoded = summed / cnt
    h2 = jax.nn.gelu(jnp.dot(decoded, Wp1_ref[...],
                             preferred_element_type=jnp.float32) + bp1_ref[...][None, :])
    out_ref[...] = jnp.dot(h2, Wp2_ref[...],
                           preferred_element_type=jnp.float32) + bp2_ref[...][None, :]


def _mlp_call(kin, fnbr, cntE, cntQ, W0, b0, W1, b1, W2, b2,
              Wp1, bp1, Wp2, bp2, Bq=256, interpret=False):
    Nq = cntQ.shape[0]
    BE = Bq * _K
    grid = (Nq // Bq,)
    full = lambda shape: pl.BlockSpec(shape, lambda i: (0,) * len(shape))
    return pl.pallas_call(
        _mlp_block,
        grid=grid,
        in_specs=[
            pl.BlockSpec((BE, 4), lambda i: (i, 0)),
            pl.BlockSpec((BE, 128), lambda i: (i, 0)),
            pl.BlockSpec((BE, 1), lambda i: (i, 0)),
            pl.BlockSpec((Bq, 1), lambda i: (i, 0)),
            full((4, 64)), full((64,)), full((64, 64)), full((64,)),
            full((64, 128)), full((128,)), full((128, 256)), full((256,)),
            full((256, 128)), full((128,)),
        ],
        out_specs=pl.BlockSpec((Bq, 128), lambda i: (i, 0)),
        out_shape=jax.ShapeDtypeStruct((Nq, 128), jnp.float32),
        interpret=interpret,
    )(kin, fnbr, cntE, cntQ, W0, b0, W1, b1, W2, b2, Wp1, bp1, Wp2, bp2)


def kernel(rndata_flat, phys_pos_query, batch_idx_phys_query,
           latent_tokens_pos, latent_tokens_batch_idx,
           W0, b0, W1, b1, W2, b2, Wp1, bp1, Wp2, bp2):
    Nq = phys_pos_query.shape[0]
    E = Nq * _K
    latx = latent_tokens_pos[:, 0]
    laty = latent_tokens_pos[:, 1]
    qx = phys_pos_query[:, 0]
    qy = phys_pos_query[:, 1]
    nbr_flat, ynx, yny, cntQ, fnbr = _sc_search(latx, laty, qx, qy,
                                                rndata_flat)
    cntQ = cntQ[:, None]
    cntE = jnp.broadcast_to(cntQ[:, None, :], (Nq, _K, 1)).reshape(E, 1)
    x_rep = jnp.broadcast_to(phys_pos_query[:, None, :], (Nq, _K, 2))
    kin = jnp.concatenate(
        [ynx[:, None], yny[:, None], x_rep.reshape(E, 2)], axis=1)
    return _mlp_call(kin, fnbr, cntE, cntQ, W0, b0, W1, b1, W2, b2,
                     Wp1, bp1, Wp2, bp2)


# speedup vs baseline: 11.8901x; 11.8901x over previous
import functools

import jax
import jax.numpy as jnp
from jax import lax
from jax.experimental import pallas as pl
from jax.experimental.pallas import tpu as pltpu
from jax.experimental.pallas import tpu_sc as plsc

_RADIUS = 0.033
_K = 32
_G = 30
_NCELL = _G * _G
_CAP = 24
_NCELL_PAD = 912
_NQ = 16384
_NL = 4096
_NW = 32
_QPW = _NQ // _NW


_CH = 128


def _search_body(latx_hbm, laty_hbm, qx_hbm, qy_hbm, rn_hbm,
                 nbr_hbm, ynx_hbm, yny_hbm, cnt_hbm, fnbr_hbm,
                 latx, laty, ccnt, table, qx, qy,
                 nbrb, ynxb, ynyb, cntb, rows0, rows1, idx0, idx1,
                 sem0, sem1):
    wid = lax.axis_index("s") * 2 + lax.axis_index("c")
    qbase = wid * _QPW
    pltpu.sync_copy(latx_hbm, latx)
    pltpu.sync_copy(laty_hbm, laty)
    pltpu.sync_copy(qx_hbm.at[pl.ds(qbase, _QPW)], qx)
    pltpu.sync_copy(qy_hbm.at[pl.ds(qbase, _QPW)], qy)

    zi = jnp.zeros((16,), jnp.int32)
    zf = jnp.zeros((16,), jnp.float32)

    def zero_ccnt(i, _):
        ccnt[pl.ds(i * 16, 16)] = zi
        return 0
    lax.fori_loop(0, _NCELL_PAD // 16, zero_ccnt, 0)

    lane0 = lax.broadcasted_iota(jnp.int32, (16,), 0)

    def zero_bufs(i, _):
        pad = (i * 16 + lane0 + wid * 123) & (_NL - 1)
        nbrb[pl.ds(i * 16, 16)] = pad
        ynxb[pl.ds(i * 16, 16)] = zf
        ynyb[pl.ds(i * 16, 16)] = zf
        return 0
    lax.fori_loop(0, (_QPW * _K) // 16, zero_bufs, 0)

    lane = lax.broadcasted_iota(jnp.int32, (16,), 0)

    def build16(i, _):
        xv = latx[pl.ds(i * 16, 16)]
        yv = laty[pl.ds(i * 16, 16)]
        cx = (xv * _G).astype(jnp.int32)
        cy = (yv * _G).astype(jnp.int32)
        c = cy * _G + cx
        idxv = i * 16 + lane
        for j in range(16):
            m = lane == j
            cur = plsc.load_gather(ccnt, [c], mask=m)
            ok = m & (cur < _CAP)
            slot = jnp.minimum(cur, _CAP - 1)
            plsc.store_scatter(table, [c * _CAP + slot], idxv, mask=ok)
            plsc.store_scatter(ccnt, [c], cur + 1, mask=ok)
        return 0
    lax.fori_loop(0, _NL // 16, build16, 0)

    r2 = jnp.float32(_RADIUS * _RADIUS)

    def group(g, _):
        qxv = qx[pl.ds(g * 16, 16)]
        qyv = qy[pl.ds(g * 16, 16)]
        cx = (qxv * _G).astype(jnp.int32)
        cy = (qyv * _G).astype(jnp.int32)
        base = (g * 16 + lane) * _K
        nc = jnp.zeros((16,), jnp.int32)
        for dy in (-1, 0, 1):
            for dx in (-1, 0, 1):
                ny = cy + dy
                nx = cx + dx
                cellok = (ny >= 0) & (ny < _G) & (nx >= 0) & (nx < _G)
                cidx = jnp.where(cellok, ny * _G + nx, 0)
                ccv = plsc.load_gather(ccnt, [cidx], mask=cellok)
                ccv = jnp.where(cellok, ccv, 0)
                mx = jnp.max(ccv)

                def entry(e, nc):
                    valid = e < ccv
                    v = plsc.load_gather(table, [cidx * _CAP + e], mask=valid)
                    lx = plsc.load_gather(latx, [v], mask=valid)
                    ly = plsc.load_gather(laty, [v], mask=valid)
                    d0 = qxv - lx
                    d1 = qyv - ly
                    dd = d0 * d0 + d1 * d1
                    hit = valid & (dd <= r2)
                    ok = hit & (nc < _K)
                    addr = base + jnp.minimum(nc, _K - 1)
                    plsc.store_scatter(nbrb, [addr], v, mask=ok)
                    plsc.store_scatter(ynxb, [addr], lx, mask=ok)
                    plsc.store_scatter(ynyb, [addr], ly, mask=ok)
                    return nc + ok.astype(jnp.int32)

                nc = lax.fori_loop(0, mx, entry, nc)
        cntb[pl.ds(g * 16, 16)] = nc.astype(jnp.float32)
        return 0
    lax.fori_loop(0, _QPW // 16, group, 0)

    pltpu.sync_copy(nbrb, nbr_hbm.at[pl.ds(qbase * _K, _QPW * _K)])
    pltpu.sync_copy(ynxb, ynx_hbm.at[pl.ds(qbase * _K, _QPW * _K)])
    pltpu.sync_copy(ynyb, yny_hbm.at[pl.ds(qbase * _K, _QPW * _K)])
    pltpu.sync_copy(cntb, cnt_hbm.at[pl.ds(qbase, _QPW)])

    ebase = qbase * _K
    nch = (_QPW * _K) // _CH
    rows = (rows0, rows1)
    idxs = (idx0, idx1)
    sems = (sem0, sem1)
    for v in range(_CH // 16):
        idx0[pl.ds(v * 16, 16)] = nbrb[pl.ds(v * 16, 16)]
    prev = None
    for c in range(nch):
        buf = rows[c % 2]
        cp = pltpu.async_copy(rn_hbm.at[idxs[c % 2]], buf, sems[c % 2])
        if prev is not None:
            pcp, pbuf, poff = prev
            pcp.wait()
            pltpu.sync_copy(pbuf, fnbr_hbm.at[pl.ds(ebase + poff, _CH)])
        if c + 1 < nch:
            nidx = idxs[(c + 1) % 2]
            off = (c + 1) * _CH
            for v in range(_CH // 16):
                nidx[pl.ds(v * 16, 16)] = nbrb[pl.ds(off + v * 16, 16)]
        prev = (cp, buf, c * _CH)
    pcp, pbuf, poff = prev
    pcp.wait()
    pltpu.sync_copy(pbuf, fnbr_hbm.at[pl.ds(ebase + poff, _CH)])


def _sc_search(latx, laty, qx, qy, rndata_flat):
    mesh = plsc.VectorSubcoreMesh(core_axis_name="c", subcore_axis_name="s")
    f = pl.kernel(
        _search_body,
        out_type=[
            jax.ShapeDtypeStruct((_NQ * _K,), jnp.int32),
            jax.ShapeDtypeStruct((_NQ * _K,), jnp.float32),
            jax.ShapeDtypeStruct((_NQ * _K,), jnp.float32),
            jax.ShapeDtypeStruct((_NQ,), jnp.float32),
            jax.ShapeDtypeStruct((_NQ * _K, 128), jnp.float32),
        ],
        mesh=mesh,
        compiler_params=pltpu.CompilerParams(needs_layout_passes=False),
        scratch_types=[
            pltpu.VMEM((_NL,), jnp.float32),
            pltpu.VMEM((_NL,), jnp.float32),
            pltpu.VMEM((_NCELL_PAD,), jnp.int32),
            pltpu.VMEM((_NCELL * _CAP,), jnp.int32),
            pltpu.VMEM((_QPW,), jnp.float32),
            pltpu.VMEM((_QPW,), jnp.float32),
            pltpu.VMEM((_QPW * _K,), jnp.int32),
            pltpu.VMEM((_QPW * _K,), jnp.float32),
            pltpu.VMEM((_QPW * _K,), jnp.float32),
            pltpu.VMEM((_QPW,), jnp.float32),
            pltpu.VMEM((_CH, 128), jnp.float32),
            pltpu.VMEM((_CH, 128), jnp.float32),
            pltpu.VMEM((_CH,), jnp.int32),
            pltpu.VMEM((_CH,), jnp.int32),
            pltpu.SemaphoreType.DMA,
            pltpu.SemaphoreType.DMA,
        ],
    )
    return f(latx, laty, qx, qy, rndata_flat)


def _mlp_block(kin_ref, fnbr_ref, cntE_ref, cntQ_ref,
               W0_ref, b0_ref, W1_ref, b1_ref, W2_ref, b2_ref,
               Wp1_ref, bp1_ref, Wp2_ref, bp2_ref, out_ref):
    E = kin_ref.shape[0]
    Bq = E // _K
    h = jax.nn.gelu(jnp.dot(kin_ref[...], W0_ref[...],
                            preferred_element_type=jnp.float32) + b0_ref[...][None, :])
    h = jax.nn.gelu(jnp.dot(h, W1_ref[...],
                            preferred_element_type=jnp.float32) + b1_ref[...][None, :])
    kern = jnp.dot(h, W2_ref[...],
                   preferred_element_type=jnp.float32) + b2_ref[...][None, :]
    k2d = jax.lax.broadcasted_iota(jnp.int32, (E, 128), 0) % _K
    cntEb = jnp.broadcast_to(cntE_ref[...], (E, 128)).astype(jnp.int32)
    maskE = (k2d < cntEb).astype(jnp.float32)
    prod = kern * fnbr_ref[...] * maskE
    summed = prod.reshape(Bq, _K, 128).sum(axis=1)
    cnt = jnp.maximum(cntQ_ref[...], 1.0)
    decoded = summed / cnt
    h2 = jax.nn.gelu(jnp.dot(decoded, Wp1_ref[...],
                             preferred_element_type=jnp.float32) + bp1_ref[...][None, :])
    out_ref[...] = jnp.dot(h2, Wp2_ref[...],
                           preferred_element_type=jnp.float32) + bp2_ref[...][None, :]


def _mlp_call(kin, fnbr, cntE, cntQ, W0, b0, W1, b1, W2, b2,
              Wp1, bp1, Wp2, bp2, Bq=256, interpret=False):
    Nq = cntQ.shape[0]
    BE = Bq * _K
    grid = (Nq // Bq,)
    full = lambda shape: pl.BlockSpec(shape, lambda i: (0,) * len(shape))
    return pl.pallas_call(
        _mlp_block,
        grid=grid,
        in_specs=[
            pl.BlockSpec((BE, 4), lambda i: (i, 0)),
            pl.BlockSpec((BE, 128), lambda i: (i, 0)),
            pl.BlockSpec((BE, 1), lambda i: (i, 0)),
            pl.BlockSpec((Bq, 1), lambda i: (i, 0)),
            full((4, 64)), full((64,)), full((64, 64)), full((64,)),
            full((64, 128)), full((128,)), full((128, 256)), full((256,)),
            full((256, 128)), full((128,)),
        ],
        out_specs=pl.BlockSpec((Bq, 128), lambda i: (i, 0)),
        out_shape=jax.ShapeDtypeStruct((Nq, 128), jnp.float32),
        interpret=interpret,
    )(kin, fnbr, cntE, cntQ, W0, b0, W1, b1, W2, b2, Wp1, bp1, Wp2, bp2)


def kernel(rndata_flat, phys_pos_query, batch_idx_phys_query,
           latent_tokens_pos, latent_tokens_batch_idx,
           W0, b0, W1, b1, W2, b2, Wp1, bp1, Wp2, bp2):
    Nq = phys_pos_query.shape[0]
    E = Nq * _K
    latx = latent_tokens_pos[:, 0]
    laty = latent_tokens_pos[:, 1]
    qx = phys_pos_query[:, 0]
    qy = phys_pos_query[:, 1]
    nbr_flat, ynx, yny, cntQ, fnbr = _sc_search(latx, laty, qx, qy,
                                                rndata_flat)
    cntQ = cntQ[:, None]
    cntE = jnp.broadcast_to(cntQ[:, None, :], (Nq, _K, 1)).reshape(E, 1)
    x_rep = jnp.broadcast_to(phys_pos_query[:, None, :], (Nq, _K, 2))
    kin = jnp.concatenate(
        [ynx[:, None], yny[:, None], x_rep.reshape(E, 2)], axis=1)
    return _mlp_call(kin, fnbr, cntE, cntQ, W0, b0, W1, b1, W2, b2,
                     Wp1, bp1, Wp2, bp2)


# speedup vs baseline: 14.5730x; 1.2256x over previous
import functools

import jax
import jax.numpy as jnp
from jax import lax
from jax.experimental import pallas as pl
from jax.experimental.pallas import tpu as pltpu
from jax.experimental.pallas import tpu_sc as plsc

_RADIUS = 0.033
_K = 32
_G = 30
_NCELL = _G * _G
_CAP = 24
_NCELL_PAD = 912
_NQ = 16384
_NL = 4096
_NZPAD = 256
_NW = 32
_QPW = _NQ // _NW


_CH = 128


def _search_body(latx_hbm, laty_hbm, qx_hbm, qy_hbm, rn_hbm,
                 nbr_hbm, ynx_hbm, yny_hbm, cnt_hbm, fnbr_hbm,
                 latx, laty, ccnt, table, qx, qy,
                 nbrb, ynxb, ynyb, cntb, rows0, rows1, idx0, idx1,
                 sem0, sem1):
    wid = lax.axis_index("s") * 2 + lax.axis_index("c")
    qbase = wid * _QPW
    pltpu.sync_copy(latx_hbm, latx)
    pltpu.sync_copy(laty_hbm, laty)
    pltpu.sync_copy(qx_hbm.at[pl.ds(qbase, _QPW)], qx)
    pltpu.sync_copy(qy_hbm.at[pl.ds(qbase, _QPW)], qy)

    zi = jnp.zeros((16,), jnp.int32)
    zf = jnp.zeros((16,), jnp.float32)

    def zero_ccnt(i, _):
        ccnt[pl.ds(i * 16, 16)] = zi
        return 0
    lax.fori_loop(0, _NCELL_PAD // 16, zero_ccnt, 0)

    lane0 = lax.broadcasted_iota(jnp.int32, (16,), 0)

    def zero_bufs(i, _):
        pad = _NL + ((i * 16 + lane0 + wid * 123) & (_NZPAD - 1))
        nbrb[pl.ds(i * 16, 16)] = pad
        ynxb[pl.ds(i * 16, 16)] = zf
        ynyb[pl.ds(i * 16, 16)] = zf
        return 0
    lax.fori_loop(0, (_QPW * _K) // 16, zero_bufs, 0)

    lane = lax.broadcasted_iota(jnp.int32, (16,), 0)

    def build16(i, _):
        xv = latx[pl.ds(i * 16, 16)]
        yv = laty[pl.ds(i * 16, 16)]
        cx = (xv * _G).astype(jnp.int32)
        cy = (yv * _G).astype(jnp.int32)
        c = cy * _G + cx
        idxv = i * 16 + lane
        for j in range(16):
            m = lane == j
            cur = plsc.load_gather(ccnt, [c], mask=m)
            ok = m & (cur < _CAP)
            slot = jnp.minimum(cur, _CAP - 1)
            plsc.store_scatter(table, [c * _CAP + slot], idxv, mask=ok)
            plsc.store_scatter(ccnt, [c], cur + 1, mask=ok)
        return 0
    lax.fori_loop(0, _NL // 16, build16, 0)

    r2 = jnp.float32(_RADIUS * _RADIUS)

    def group(g, _):
        qxv = qx[pl.ds(g * 16, 16)]
        qyv = qy[pl.ds(g * 16, 16)]
        cx = (qxv * _G).astype(jnp.int32)
        cy = (qyv * _G).astype(jnp.int32)
        base = (g * 16 + lane) * _K
        nc = jnp.zeros((16,), jnp.int32)
        for dy in (-1, 0, 1):
            for dx in (-1, 0, 1):
                ny = cy + dy
                nx = cx + dx
                cellok = (ny >= 0) & (ny < _G) & (nx >= 0) & (nx < _G)
                cidx = jnp.where(cellok, ny * _G + nx, 0)
                ccv = plsc.load_gather(ccnt, [cidx], mask=cellok)
                ccv = jnp.where(cellok, ccv, 0)
                mx = jnp.max(ccv)

                def entry(e, nc):
                    valid = e < ccv
                    v = plsc.load_gather(table, [cidx * _CAP + e], mask=valid)
                    lx = plsc.load_gather(latx, [v], mask=valid)
                    ly = plsc.load_gather(laty, [v], mask=valid)
                    d0 = qxv - lx
                    d1 = qyv - ly
                    dd = d0 * d0 + d1 * d1
                    hit = valid & (dd <= r2)
                    ok = hit & (nc < _K)
                    addr = base + jnp.minimum(nc, _K - 1)
                    plsc.store_scatter(nbrb, [addr], v, mask=ok)
                    plsc.store_scatter(ynxb, [addr], lx, mask=ok)
                    plsc.store_scatter(ynyb, [addr], ly, mask=ok)
                    return nc + ok.astype(jnp.int32)

                nc = lax.fori_loop(0, mx, entry, nc)
        cntb[pl.ds(g * 16, 16)] = nc.astype(jnp.float32)
        return 0
    lax.fori_loop(0, _QPW // 16, group, 0)

    pltpu.sync_copy(nbrb, nbr_hbm.at[pl.ds(qbase * _K, _QPW * _K)])
    pltpu.sync_copy(ynxb, ynx_hbm.at[pl.ds(qbase * _K, _QPW * _K)])
    pltpu.sync_copy(ynyb, yny_hbm.at[pl.ds(qbase * _K, _QPW * _K)])
    pltpu.sync_copy(cntb, cnt_hbm.at[pl.ds(qbase, _QPW)])

    ebase = qbase * _K
    nch = (_QPW * _K) // _CH
    rows = (rows0, rows1)
    idxs = (idx0, idx1)
    sems = (sem0, sem1)
    for v in range(_CH // 16):
        idx0[pl.ds(v * 16, 16)] = nbrb[pl.ds(v * 16, 16)]
    prev = None
    for c in range(nch):
        buf = rows[c % 2]
        cp = pltpu.async_copy(rn_hbm.at[idxs[c % 2]], buf, sems[c % 2])
        if prev is not None:
            pcp, pbuf, poff = prev
            pcp.wait()
            pltpu.sync_copy(pbuf, fnbr_hbm.at[pl.ds(ebase + poff, _CH)])
        if c + 1 < nch:
            nidx = idxs[(c + 1) % 2]
            off = (c + 1) * _CH
            for v in range(_CH // 16):
                nidx[pl.ds(v * 16, 16)] = nbrb[pl.ds(off + v * 16, 16)]
        prev = (cp, buf, c * _CH)
    pcp, pbuf, poff = prev
    pcp.wait()
    pltpu.sync_copy(pbuf, fnbr_hbm.at[pl.ds(ebase + poff, _CH)])


def _sc_search(latx, laty, qx, qy, rndata_flat):
    mesh = plsc.VectorSubcoreMesh(core_axis_name="c", subcore_axis_name="s")
    f = pl.kernel(
        _search_body,
        out_type=[
            jax.ShapeDtypeStruct((_NQ * _K,), jnp.int32),
            jax.ShapeDtypeStruct((_NQ * _K,), jnp.float32),
            jax.ShapeDtypeStruct((_NQ * _K,), jnp.float32),
            jax.ShapeDtypeStruct((_NQ,), jnp.float32),
            jax.ShapeDtypeStruct((_NQ * _K, 128), jnp.float32),
        ],
        mesh=mesh,
        compiler_params=pltpu.CompilerParams(needs_layout_passes=False),
        scratch_types=[
            pltpu.VMEM((_NL,), jnp.float32),
            pltpu.VMEM((_NL,), jnp.float32),
            pltpu.VMEM((_NCELL_PAD,), jnp.int32),
            pltpu.VMEM((_NCELL * _CAP,), jnp.int32),
            pltpu.VMEM((_QPW,), jnp.float32),
            pltpu.VMEM((_QPW,), jnp.float32),
            pltpu.VMEM((_QPW * _K,), jnp.int32),
            pltpu.VMEM((_QPW * _K,), jnp.float32),
            pltpu.VMEM((_QPW * _K,), jnp.float32),
            pltpu.VMEM((_QPW,), jnp.float32),
            pltpu.VMEM((_CH, 128), jnp.float32),
            pltpu.VMEM((_CH, 128), jnp.float32),
            pltpu.VMEM((_CH,), jnp.int32),
            pltpu.VMEM((_CH,), jnp.int32),
            pltpu.SemaphoreType.DMA,
            pltpu.SemaphoreType.DMA,
        ],
    )
    return f(latx, laty, qx, qy, rndata_flat)


def _mlp_block(kin_ref, fnbr_ref, cntQ_ref,
               W0_ref, b0_ref, W1_ref, b1_ref, W2_ref, b2_ref,
               Wp1_ref, bp1_ref, Wp2_ref, bp2_ref, out_ref):
    E = kin_ref.shape[0]
    Bq = E // _K
    h = jax.nn.gelu(jnp.dot(kin_ref[...], W0_ref[...],
                            preferred_element_type=jnp.float32) + b0_ref[...][None, :])
    h = jax.nn.gelu(jnp.dot(h, W1_ref[...],
                            preferred_element_type=jnp.float32) + b1_ref[...][None, :])
    kern = jnp.dot(h, W2_ref[...],
                   preferred_element_type=jnp.float32) + b2_ref[...][None, :]
    prod = kern * fnbr_ref[...]
    summed = prod.reshape(Bq, _K, 128).sum(axis=1)
    cnt = jnp.maximum(cntQ_ref[...], 1.0)
    decoded = summed / cnt
    h2 = jax.nn.gelu(jnp.dot(decoded, Wp1_ref[...],
                             preferred_element_type=jnp.float32) + bp1_ref[...][None, :])
    out_ref[...] = jnp.dot(h2, Wp2_ref[...],
                           preferred_element_type=jnp.float32) + bp2_ref[...][None, :]


def _mlp_call(kin, fnbr, cntQ, W0, b0, W1, b1, W2, b2,
              Wp1, bp1, Wp2, bp2, Bq=256, interpret=False):
    Nq = cntQ.shape[0]
    BE = Bq * _K
    grid = (Nq // Bq,)
    full = lambda shape: pl.BlockSpec(shape, lambda i: (0,) * len(shape))
    return pl.pallas_call(
        _mlp_block,
        grid=grid,
        in_specs=[
            pl.BlockSpec((BE, 4), lambda i: (i, 0)),
            pl.BlockSpec((BE, 128), lambda i: (i, 0)),
            pl.BlockSpec((Bq, 1), lambda i: (i, 0)),
            full((4, 64)), full((64,)), full((64, 64)), full((64,)),
            full((64, 128)), full((128,)), full((128, 256)), full((256,)),
            full((256, 128)), full((128,)),
        ],
        out_specs=pl.BlockSpec((Bq, 128), lambda i: (i, 0)),
        out_shape=jax.ShapeDtypeStruct((Nq, 128), jnp.float32),
        interpret=interpret,
    )(kin, fnbr, cntQ, W0, b0, W1, b1, W2, b2, Wp1, bp1, Wp2, bp2)


def kernel(rndata_flat, phys_pos_query, batch_idx_phys_query,
           latent_tokens_pos, latent_tokens_batch_idx,
           W0, b0, W1, b1, W2, b2, Wp1, bp1, Wp2, bp2):
    Nq = phys_pos_query.shape[0]
    E = Nq * _K
    latx = latent_tokens_pos[:, 0]
    laty = latent_tokens_pos[:, 1]
    qx = phys_pos_query[:, 0]
    qy = phys_pos_query[:, 1]
    rn_ext = jnp.concatenate(
        [rndata_flat, jnp.zeros((_NZPAD, 128), jnp.float32)], axis=0)
    nbr_flat, ynx, yny, cntQ, fnbr = _sc_search(latx, laty, qx, qy, rn_ext)
    cntQ = cntQ[:, None]
    x_rep = jnp.broadcast_to(phys_pos_query[:, None, :], (Nq, _K, 2))
    kin = jnp.concatenate(
        [ynx[:, None], yny[:, None], x_rep.reshape(E, 2)], axis=1)
    return _mlp_call(kin, fnbr, cntQ, W0, b0, W1, b1, W2, b2,
                     Wp1, bp1, Wp2, bp2)
